# baseline (speedup 1.0000x reference)
import functools

import jax
import jax.numpy as jnp
from jax import lax
from jax.experimental import pallas as pl
from jax.experimental.pallas import tpu as pltpu, tpu_sc as plsc

_SIGMA = 0.5
_IOU_THRESH = 0.7
_NPAD = 5120
_RT = 4096
_BR = 1024
_NC = 512
_SC_ROWS = _NPAD - _RT
_NW = 32
_RPW = _SC_ROWS // _NW
_L = 16


def _nms_body(cols_ref, rows_ref, out_ref):
    i = pl.program_id(0)
    rx1 = rows_ref[:, 0:1]
    ry1 = rows_ref[:, 1:2]
    rx2 = rows_ref[:, 2:3]
    ry2 = rows_ref[:, 3:4]
    rs = rows_ref[:, 4:5]
    r_area = (rx2 - rx1) * (ry2 - ry1)
    ridx = lax.broadcasted_iota(jnp.int32, (_BR, 1), 0) + i * _BR

    acc = jnp.zeros((_BR, 1), jnp.float32)
    for c in range(_NPAD // _NC):
        sl = slice(c * _NC, (c + 1) * _NC)
        cx1 = cols_ref[0:1, sl]
        cy1 = cols_ref[1:2, sl]
        cx2 = cols_ref[2:3, sl]
        cy2 = cols_ref[3:4, sl]
        cs = cols_ref[4:5, sl]
        c_area = (cx2 - cx1) * (cy2 - cy1)
        xx1 = jnp.maximum(rx1, cx1)
        yy1 = jnp.maximum(ry1, cy1)
        xx2 = jnp.minimum(rx2, cx2)
        yy2 = jnp.minimum(ry2, cy2)
        w = jnp.maximum(xx2 - xx1, 0.0)
        h = jnp.maximum(yy2 - yy1, 0.0)
        inter = w * h
        union = r_area + c_area - inter
        iou = inter / union
        cidx = lax.broadcasted_iota(jnp.int32, (1, _NC), 1) + c * _NC
        mask = (cs > rs) | ((cs == rs) & (cidx < ridx))
        acc = jnp.maximum(
            acc, jnp.max(jnp.where(mask, iou, 0.0), axis=1, keepdims=True))

    decay = jnp.exp(-(acc * acc) / _SIGMA)
    keep = (acc <= _IOU_THRESH).astype(jnp.float32)
    out_ref[...] = rs * decay * keep


def _sc_maxiou_kernel(x1h, y1h, x2h, y2h, sh, outh,
                      x1v, y1v, x2v, y2v, sv, rowv):
    wid = lax.axis_index("s") * 2 + lax.axis_index("c")
    base = _RT + wid * _RPW
    pltpu.sync_copy(x1h, x1v)
    pltpu.sync_copy(y1h, y1v)
    pltpu.sync_copy(x2h, x2v)
    pltpu.sync_copy(y2h, y2v)
    pltpu.sync_copy(sh, sv)

    lane = lax.iota(jnp.int32, _L)

    def group_body(g, _):
        gsl = pl.ds(base + g * _L, _L)
        rx1 = x1v[gsl]
        ry1 = y1v[gsl]
        rx2 = x2v[gsl]
        ry2 = y2v[gsl]
        rs = sv[gsl]
        r_area = (rx2 - rx1) * (ry2 - ry1)
        ridx = lane + (base + g * _L)

        def col_body(j, acc):
            cj = pl.ds(j * _L, _L)
            cvx1 = x1v[cj]
            cvy1 = y1v[cj]
            cvx2 = x2v[cj]
            cvy2 = y2v[cj]
            cvs = sv[cj]
            for cc in range(_L):
                cx1 = cvx1[cc]
                cy1 = cvy1[cc]
                cx2 = cvx2[cc]
                cy2 = cvy2[cc]
                cs = cvs[cc]
                c_area = (cx2 - cx1) * (cy2 - cy1)
                xx1 = jnp.maximum(rx1, cx1)
                yy1 = jnp.maximum(ry1, cy1)
                xx2 = jnp.minimum(rx2, cx2)
                yy2 = jnp.minimum(ry2, cy2)
                w = jnp.maximum(xx2 - xx1, 0.0)
                h = jnp.maximum(yy2 - yy1, 0.0)
                inter = w * h
                union = r_area + c_area - inter
                iou = inter / union
                cidx = j * _L + cc
                mask = (cs > rs) | ((cs == rs) & (cidx < ridx))
                acc = jnp.maximum(acc, jnp.where(mask, iou, 0.0))
            return acc

        acc = lax.fori_loop(0, _NPAD // _L, col_body,
                            jnp.zeros((_L,), jnp.float32))
        rowv[pl.ds(g * _L, _L)] = acc
        return 0

    lax.fori_loop(0, _RPW // _L, group_body, 0)
    pltpu.sync_copy(rowv, outh.at[pl.ds(wid * _RPW, _RPW)])


@jax.jit
def _nms_scores_pallas(boxes, scores):
    n = scores.shape[0]
    pad = _NPAD - n
    b = jnp.pad(boxes, ((0, pad), (0, 0)))
    s = jnp.pad(scores, (0, pad), constant_values=-1.0)
    cols = jnp.zeros((8, _NPAD), jnp.float32)
    cols = cols.at[0:4, :].set(b.T).at[4, :].set(s)
    rows = jnp.zeros((_NPAD, 8), jnp.float32)
    rows = rows.at[:, 0:4].set(b).at[:, 4].set(s)

    tc_out = pl.pallas_call(
        _nms_body,
        grid=(_RT // _BR,),
        in_specs=[
            pl.BlockSpec((8, _NPAD), lambda i: (0, 0)),
            pl.BlockSpec((_BR, 8), lambda i: (i, 0)),
        ],
        out_specs=pl.BlockSpec((_BR, 1), lambda i: (i, 0)),
        out_shape=jax.ShapeDtypeStruct((_RT, 1), jnp.float32),
    )(cols, rows)

    sc_call = pl.kernel(
        _sc_maxiou_kernel,
        out_type=jax.ShapeDtypeStruct((_SC_ROWS,), jnp.float32),
        mesh=plsc.VectorSubcoreMesh(core_axis_name="c", subcore_axis_name="s"),
        scratch_types=[
            pltpu.VMEM((_NPAD,), jnp.float32),
            pltpu.VMEM((_NPAD,), jnp.float32),
            pltpu.VMEM((_NPAD,), jnp.float32),
            pltpu.VMEM((_NPAD,), jnp.float32),
            pltpu.VMEM((_NPAD,), jnp.float32),
            pltpu.VMEM((_RPW,), jnp.float32),
        ],
    )
    sc_maxiou = sc_call(b[:, 0], b[:, 1], b[:, 2], b[:, 3], s)

    sc_s = s[_RT:]
    sc_decay = jnp.exp(-(sc_maxiou * sc_maxiou) / _SIGMA)
    sc_keep = (sc_maxiou <= _IOU_THRESH).astype(jnp.float32)
    sc_scores = sc_s * sc_decay * sc_keep
    full = jnp.concatenate([tc_out[:, 0], sc_scores])
    return full[:n]


def kernel(boxes, scores, k):
    new_scores = _nms_scores_pallas(boxes, scores)
    topk_vals, topk_idx = jax.lax.top_k(new_scores, 150)
    return new_scores, topk_vals, topk_idx


# speedup vs baseline: 1.9762x; 1.9762x over previous
import jax
import jax.numpy as jnp
from jax import lax
from jax.experimental import pallas as pl

_SIGMA = 0.5
_IOU_THRESH = 0.7
_NPAD = 5120
_BR = 1024
_NC = 512


def _nms_body(cols_ref, rows_ref, out_ref):
    i = pl.program_id(0)
    rx1 = rows_ref[:, 0:1]
    ry1 = rows_ref[:, 1:2]
    rx2 = rows_ref[:, 2:3]
    ry2 = rows_ref[:, 3:4]
    rs = rows_ref[:, 4:5]
    r_area = (rx2 - rx1) * (ry2 - ry1)
    ridx = lax.broadcasted_iota(jnp.int32, (_BR, 1), 0) + i * _BR

    acc = jnp.zeros((_BR, 1), jnp.float32)
    for c in range(_NPAD // _NC):
        sl = slice(c * _NC, (c + 1) * _NC)
        cx1 = cols_ref[0:1, sl]
        cy1 = cols_ref[1:2, sl]
        cx2 = cols_ref[2:3, sl]
        cy2 = cols_ref[3:4, sl]
        cs = cols_ref[4:5, sl]
        c_area = (cx2 - cx1) * (cy2 - cy1)
        xx1 = jnp.maximum(rx1, cx1)
        yy1 = jnp.maximum(ry1, cy1)
        xx2 = jnp.minimum(rx2, cx2)
        yy2 = jnp.minimum(ry2, cy2)
        w = jnp.maximum(xx2 - xx1, 0.0)
        h = jnp.maximum(yy2 - yy1, 0.0)
        inter = w * h
        union = r_area + c_area - inter
        iou = inter / union
        cidx = lax.broadcasted_iota(jnp.int32, (1, _NC), 1) + c * _NC
        mask = (cs > rs) | ((cs == rs) & (cidx < ridx))
        acc = jnp.maximum(
            acc, jnp.max(jnp.where(mask, iou, 0.0), axis=1, keepdims=True))

    decay = jnp.exp(-(acc * acc) / _SIGMA)
    keep = (acc <= _IOU_THRESH).astype(jnp.float32)
    out_ref[...] = rs * decay * keep


@jax.jit
def _nms_scores_pallas(boxes, scores):
    n = scores.shape[0]
    pad = _NPAD - n
    b = jnp.pad(boxes, ((0, pad), (0, 0)))
    s = jnp.pad(scores, (0, pad), constant_values=-1.0)
    cols = jnp.zeros((8, _NPAD), jnp.float32)
    cols = cols.at[0:4, :].set(b.T).at[4, :].set(s)
    rows = jnp.zeros((_NPAD, 8), jnp.float32)
    rows = rows.at[:, 0:4].set(b).at[:, 4].set(s)

    out = pl.pallas_call(
        _nms_body,
        grid=(_NPAD // _BR,),
        in_specs=[
            pl.BlockSpec((8, _NPAD), lambda i: (0, 0)),
            pl.BlockSpec((_BR, 8), lambda i: (i, 0)),
        ],
        out_specs=pl.BlockSpec((_BR, 1), lambda i: (i, 0)),
        out_shape=jax.ShapeDtypeStruct((_NPAD, 1), jnp.float32),
    )(cols, rows)
    return out[:n, 0]


def kernel(boxes, scores, k):
    new_scores = _nms_scores_pallas(boxes, scores)
    topk_vals, topk_idx = jax.lax.top_k(new_scores, 150)
    return new_scores, topk_vals, topk_idx
